# baseline (speedup 1.0000x reference)
import functools

import jax
import jax.numpy as jnp
from jax import lax
from jax.experimental import pallas as pl
from jax.experimental.pallas import tpu as pltpu
from jax.experimental.pallas import tpu_sc as plsc

BATCH = 4
SEQ = 2048
D = 768
LANES = 16
VECS_PER_ROW = D // LANES

_INFO = plsc.get_sparse_core_info()
NUM_CORES = _INFO.num_cores
NUM_SUBCORES = _INFO.num_subcores
NW = NUM_CORES * NUM_SUBCORES
ROWS_PER_W = BATCH * SEQ // NW
CHUNK = 64
NCHUNK = ROWS_PER_W // CHUNK


def _emb_body(x_hbm, tok_hbm, pos_hbm, out_hbm, idx_v, tok_v, pos_v, sem):
    wid = lax.axis_index("s") * NUM_CORES + lax.axis_index("c")
    base = wid * ROWS_PER_W
    pos_base = lax.rem(base, SEQ)
    pltpu.sync_copy(x_hbm.at[pl.ds(base, ROWS_PER_W)], idx_v)

    def chunk_body(c, carry):
        row0 = c * CHUNK
        gather = pltpu.async_copy(
            tok_hbm.at[idx_v.at[pl.ds(row0, CHUNK)]], tok_v, sem)
        pltpu.sync_copy(pos_hbm.at[pl.ds(pos_base + row0, CHUNK)], pos_v)
        gather.wait()

        def row_body(r, carry_r):
            def vec_body(j, carry_j):
                sl = pl.ds(j * LANES, LANES)
                plsc.addupdate(pos_v.at[r, sl], tok_v[r, sl])
                return carry_j
            return lax.fori_loop(0, VECS_PER_ROW, vec_body, carry_r)

        lax.fori_loop(0, CHUNK, row_body, 0)
        pltpu.sync_copy(pos_v, out_hbm.at[pl.ds(base + row0, CHUNK)])
        return carry

    lax.fori_loop(0, NCHUNK, chunk_body, 0)


@functools.partial(
    pl.kernel,
    out_type=jax.ShapeDtypeStruct((BATCH * SEQ, D), jnp.float32),
    mesh=plsc.VectorSubcoreMesh(core_axis_name="c", subcore_axis_name="s"),
    scratch_types=[
        pltpu.VMEM((ROWS_PER_W,), jnp.int32),
        pltpu.VMEM((CHUNK, D), jnp.float32),
        pltpu.VMEM((CHUNK, D), jnp.float32),
        pltpu.SemaphoreType.DMA,
    ],
)
def _emb_kernel(x_hbm, tok_hbm, pos_hbm, out_hbm, idx_v, tok_v, pos_v, sem):
    _emb_body(x_hbm, tok_hbm, pos_hbm, out_hbm, idx_v, tok_v, pos_v, sem)


def kernel(x, token_table, pos_table):
    flat_idx = x.reshape(-1).astype(jnp.int32)
    out = _emb_kernel(flat_idx, token_table, pos_table)
    return out.reshape(BATCH, SEQ, D)


# speedup vs baseline: 1.8734x; 1.8734x over previous
import functools

import jax
import jax.numpy as jnp
from jax import lax
from jax.experimental import pallas as pl
from jax.experimental.pallas import tpu as pltpu
from jax.experimental.pallas import tpu_sc as plsc

BATCH = 4
SEQ = 2048
D = 768
LANES = 16
VECS_PER_ROW = D // LANES

_INFO = plsc.get_sparse_core_info()
NUM_CORES = _INFO.num_cores
NUM_SUBCORES = _INFO.num_subcores
NW = NUM_CORES * NUM_SUBCORES
ROWS_PER_W = BATCH * SEQ // NW
CHUNK = 32
NCHUNK = ROWS_PER_W // CHUNK
NTOK = 3
NPOS = 2


def _emb_body(x_hbm, tok_hbm, pos_hbm, out_hbm, idx_v,
              tok_bufs, pos_bufs, gsems, psems, osems):
    wid = lax.axis_index("s") * NUM_CORES + lax.axis_index("c")
    base = wid * ROWS_PER_W
    pos_base = lax.rem(base, SEQ)
    pltpu.sync_copy(x_hbm.at[pl.ds(base, ROWS_PER_W)], idx_v)

    def issue_gather(c):
        return pltpu.async_copy(
            tok_hbm.at[idx_v.at[pl.ds(c * CHUNK, CHUNK)]],
            tok_bufs[c % NTOK], gsems[c % NTOK])

    def issue_pos(c):
        return pltpu.async_copy(
            pos_hbm.at[pl.ds(pos_base + c * CHUNK, CHUNK)],
            pos_bufs[c % NPOS], psems[c % NPOS])

    def issue_out(c):
        return pltpu.async_copy(
            tok_bufs[c % NTOK],
            out_hbm.at[pl.ds(base + c * CHUNK, CHUNK)], osems[c % NTOK])

    gathers = [issue_gather(0), issue_gather(1)]
    poss = [issue_pos(0), issue_pos(1)]
    outs = []
    out_waited = set()
    for c in range(NCHUNK):
        gathers[c].wait()
        poss[c].wait()
        tok_v = tok_bufs[c % NTOK]
        pos_v = pos_bufs[c % NPOS]

        def row_body(r, carry, tok_v=tok_v, pos_v=pos_v):
            for j in range(VECS_PER_ROW):
                sl = pl.ds(j * LANES, LANES)
                plsc.addupdate(tok_v.at[r, sl], pos_v[r, sl])
            return carry

        lax.fori_loop(0, CHUNK, row_body, 0)
        if c + 2 < NCHUNK:
            poss.append(issue_pos(c + 2))
        outs.append(issue_out(c))
        if c + 2 < NCHUNK:
            if c >= 1:
                outs[c - 1].wait()
                out_waited.add(c - 1)
            gathers.append(issue_gather(c + 2))
    for c in range(NCHUNK):
        if c not in out_waited:
            outs[c].wait()


@functools.partial(
    pl.kernel,
    out_type=jax.ShapeDtypeStruct((BATCH * SEQ, D), jnp.float32),
    mesh=plsc.VectorSubcoreMesh(core_axis_name="c", subcore_axis_name="s"),
    scratch_types=[
        pltpu.VMEM((ROWS_PER_W,), jnp.int32),
        [pltpu.VMEM((CHUNK, D), jnp.float32) for _ in range(NTOK)],
        [pltpu.VMEM((CHUNK, D), jnp.float32) for _ in range(NPOS)],
        [pltpu.SemaphoreType.DMA for _ in range(NTOK)],
        [pltpu.SemaphoreType.DMA for _ in range(NPOS)],
        [pltpu.SemaphoreType.DMA for _ in range(NTOK)],
    ],
)
def _emb_kernel(x_hbm, tok_hbm, pos_hbm, out_hbm, idx_v,
                tok_bufs, pos_bufs, gsems, psems, osems):
    _emb_body(x_hbm, tok_hbm, pos_hbm, out_hbm, idx_v,
              tok_bufs, pos_bufs, gsems, psems, osems)


def kernel(x, token_table, pos_table):
    flat_idx = x.reshape(-1).astype(jnp.int32)
    out = _emb_kernel(flat_idx, token_table, pos_table)
    return out.reshape(BATCH, SEQ, D)
